# baseline (speedup 1.0000x reference)
import functools

import jax
import jax.numpy as jnp
from jax import lax
from jax.experimental import pallas as pl
from jax.experimental.pallas import tpu as pltpu
from jax.experimental.pallas import tpu_sc as plsc

C = 128
K = 32
NC = 2
NS = 16
NW = NC * NS
L = 16
CH = 8
PTS = 320
CAP = PTS * (K // CH)
CPD = 8
NBUF = 6
CAPM = 1296
BSTR = 128
NEG = -3.4e38


def _pool_body(table_hbm, idx_hbm, meta_hbm, glen_hbm, out_hbm,
               idx_v, meta_v, glen_v, bld_v, rows_v, acc_v, gsems):
    wid = lax.axis_index("s") * NC + lax.axis_index("c")

    pltpu.sync_copy(idx_hbm.at[pl.ds(wid * (PTS * K), PTS * K)],
                    idx_v.at[pl.ds(0, PTS * K)])
    pltpu.sync_copy(meta_hbm.at[pl.ds(wid * (CAPM * 8), CAPM * 8)], meta_v)
    pltpu.sync_copy(glen_hbm.at[pl.ds(wid * 128, 128)], glen_v)

    nring = glen_v[pl.ds(0, L)][0]
    n_groups = nring * NBUF

    neg = jnp.full((L,), NEG, jnp.float32)

    def init_row(r, _):
        for f in range(C // L):
            acc_v[pl.ds(r * C + f * L, L)] = neg
        return ()

    lax.fori_loop(0, PTS + 1, init_row, (), unroll=4)

    def build(g, b):
        def bchunk(k, _, b=b, g=g):
            mv = meta_v[pl.ds((g * CPD + k) * 8, L)]
            src = mv[0] >> 10
            bld_v[pl.ds(b * BSTR + k * CH, L)] = idx_v[pl.ds(src, L)]
            return ()

        lax.fori_loop(0, CPD, bchunk, (), unroll=8)

    def gather(g, b):
        return pltpu.make_async_copy(
            table_hbm.at[bld_v.at[pl.ds(b * BSTR, CPD * CH)]],
            rows_v.at[b], gsems.at[b])

    for b in range(NBUF):
        build(b, b)
        gather(b, b).start()

    def ring(i, _):
        g0 = i * NBUF
        for b in range(NBUF):
            g = g0 + b
            gather(g, b).wait()

            def chunk(k, _, b=b, g=g):
                def row_step(r, accs, b=b, k=k):
                    return tuple(
                        jnp.maximum(accs[f],
                                    rows_v[b, k * CH + r, pl.ds(f * L, L)])
                        for f in range(C // L))

                accs0 = tuple(rows_v[b, k * CH, pl.ds(f * L, L)]
                              for f in range(C // L))
                accs = lax.fori_loop(1, CH, row_step, accs0, unroll=7)

                mv = meta_v[pl.ds((g * CPD + k) * 8, L)]
                pt = mv[0] & 1023
                rowbase = pt * C
                for f in range(C // L):
                    off = rowbase + f * L
                    acc_v[pl.ds(off, L)] = jnp.maximum(
                        acc_v[pl.ds(off, L)], accs[f])
                return ()

            lax.fori_loop(0, CPD, chunk, (), unroll=4)

            @pl.when(g + NBUF < n_groups)
            def _(b=b, g=g):
                build(g + NBUF, b)
                gather(g + NBUF, b).start()

        return ()

    lax.fori_loop(0, nring, ring, (), unroll=False)

    pltpu.sync_copy(acc_v.at[pl.ds(0, PTS * C)],
                    out_hbm.at[pl.ds(wid * (PTS * C), PTS * C)])


def kernel(inputs, nn_count, nn_index):
    mp = nn_count.shape[0]
    mp_pad = NW * PTS

    idx = nn_index.astype(jnp.int32)
    count = nn_count.astype(jnp.int32)
    mask = jnp.arange(K, dtype=jnp.int32)[None, :] < count[:, None]
    idx_dup = jnp.where(mask, idx, idx[:, :1])
    idx_pad = jnp.zeros((mp_pad, K), jnp.int32).at[:mp].set(idx_dup)
    count_pad = jnp.zeros((mp_pad,), jnp.int32).at[:mp].set(count)

    nb = (count_pad + (CH - 1)) // CH
    nb_w = nb.reshape(NW, PTS)
    cs = jnp.cumsum(nb_w, axis=1)
    total = cs[:, -1]
    pos = jnp.arange(CAP, dtype=jnp.int32)
    widx = jnp.broadcast_to(jnp.arange(NW, dtype=jnp.int32)[:, None],
                            (NW, PTS))
    ind = jnp.zeros((NW, CAP + 1), jnp.int32).at[
        widx, jnp.minimum(cs, CAP)].add(1)
    pt = jnp.cumsum(ind[:, :CAP], axis=1)
    pt_c = jnp.minimum(pt, PTS - 1).astype(jnp.int32)
    cs_excl = cs - nb_w
    sspos = jnp.where(nb_w > 0, cs_excl, CAP)
    starts = jnp.zeros((NW, CAP + 1), jnp.int32).at[
        widx, jnp.minimum(sspos, CAP)].add(cs_excl)
    start = jax.lax.cummax(starts[:, :CAP], axis=1)
    sub = (pos[None, :] - start).astype(jnp.int32)
    valid = pos[None, :] < total[:, None]

    csrc = pt_c * K + sub * CH
    meta = jnp.where(valid, (csrc << 10) | pt_c, PTS)
    meta = jnp.concatenate(
        [meta, jnp.full((NW, CAPM - CAP), PTS, jnp.int32)], axis=1)
    meta_rep = jnp.broadcast_to(meta[:, :, None], (NW, CAPM, 8)).reshape(-1)
    glen = ((total + CPD * NBUF - 1) // (CPD * NBUF)).astype(jnp.int32)
    glen_rep = jnp.broadcast_to(glen[:, None], (NW, 128)).reshape(-1)

    grid_kernel = pl.kernel(
        _pool_body,
        out_type=jax.ShapeDtypeStruct((mp_pad * C,), jnp.float32),
        mesh=plsc.VectorSubcoreMesh(core_axis_name="c", subcore_axis_name="s"),
        scratch_types=[
            pltpu.VMEM((PTS * K + 8,), jnp.int32),
            pltpu.VMEM((CAPM * 8,), jnp.int32),
            pltpu.VMEM((128,), jnp.int32),
            pltpu.VMEM((NBUF * BSTR,), jnp.int32),
            pltpu.VMEM((NBUF, CPD * CH, C), jnp.float32),
            pltpu.VMEM(((PTS + 1) * C,), jnp.float32),
            pltpu.SemaphoreType.DMA((NBUF,)),
        ],
    )
    out = grid_kernel(inputs, idx_pad.reshape(-1), meta_rep, glen_rep)
    return out.reshape(mp_pad, C)[:mp]


# speedup vs baseline: 1.1357x; 1.1357x over previous
import functools

import jax
import jax.numpy as jnp
from jax import lax
from jax.experimental import pallas as pl
from jax.experimental.pallas import tpu as pltpu
from jax.experimental.pallas import tpu_sc as plsc

C = 128
K = 32
NC = 2
NS = 16
NW = NC * NS
L = 16
CH = 8
PTS = 320
CAP = PTS * (K // CH)
CPD = 8
NBUF = 6
CAPM = 1296
BSTR = 128
NEG = -3.4e38


def _pool_body(table_hbm, idx_hbm, meta_hbm, glen_hbm, out_hbm,
               idx_v, meta_v, glen_v, bld_v, rows_v, acc_v, gsems):
    wid = lax.axis_index("s") * NC + lax.axis_index("c")

    pltpu.sync_copy(idx_hbm.at[pl.ds(wid * (PTS * K), PTS * K)],
                    idx_v.at[pl.ds(0, PTS * K)])
    pltpu.sync_copy(meta_hbm.at[pl.ds(wid * (CAPM * 8), CAPM * 8)], meta_v)
    pltpu.sync_copy(glen_hbm.at[pl.ds(wid * 128, 128)], glen_v)

    nring = glen_v[pl.ds(0, L)][0]
    n_groups = nring * NBUF

    neg = jnp.full((L,), NEG, jnp.float32)

    def init_row(r, _):
        for f in range(C // L):
            acc_v[pl.ds(r * C + f * L, L)] = neg
        return ()

    lax.fori_loop(0, PTS + 1, init_row, (), unroll=4)

    def build(g, b):
        def bchunk(k, _, b=b, g=g):
            mv = meta_v[pl.ds((g * CPD + k) * 8, L)]
            src = mv[0] >> 10
            bld_v[pl.ds(b * BSTR + k * CH, L)] = idx_v[pl.ds(src, L)]
            return ()

        lax.fori_loop(0, CPD, bchunk, (), unroll=4)

    def gather(g, b):
        return pltpu.make_async_copy(
            table_hbm.at[bld_v.at[pl.ds(b * BSTR, CPD * CH)]],
            rows_v.at[b], gsems.at[b])

    for b in range(NBUF):
        build(b, b)
        gather(b, b).start()

    def ring(i, _):
        g0 = i * NBUF
        for b in range(NBUF):
            g = g0 + b
            gather(g, b).wait()

            def chunk(k, _, b=b, g=g):
                def row_step(r, accs, b=b, k=k):
                    return tuple(
                        jnp.maximum(accs[f],
                                    rows_v[b, k * CH + r, pl.ds(f * L, L)])
                        for f in range(C // L))

                accs0 = tuple(rows_v[b, k * CH, pl.ds(f * L, L)]
                              for f in range(C // L))
                accs = lax.fori_loop(1, CH, row_step, accs0, unroll=7)

                mv = meta_v[pl.ds((g * CPD + k) * 8, L)]
                pt = mv[0] & 1023
                rowbase = pt * C
                for f in range(C // L):
                    off = rowbase + f * L
                    acc_v[pl.ds(off, L)] = jnp.maximum(
                        acc_v[pl.ds(off, L)], accs[f])
                return ()

            lax.fori_loop(0, CPD, chunk, (), unroll=1)

            @pl.when(g + NBUF < n_groups)
            def _(b=b, g=g):
                build(g + NBUF, b)
                gather(g + NBUF, b).start()

        return ()

    lax.fori_loop(0, nring, ring, (), unroll=False)

    pltpu.sync_copy(acc_v.at[pl.ds(0, PTS * C)],
                    out_hbm.at[pl.ds(wid * (PTS * C), PTS * C)])


def kernel(inputs, nn_count, nn_index):
    mp = nn_count.shape[0]
    mp_pad = NW * PTS

    idx = nn_index.astype(jnp.int32)
    count = nn_count.astype(jnp.int32)
    mask = jnp.arange(K, dtype=jnp.int32)[None, :] < count[:, None]
    idx_dup = jnp.where(mask, idx, idx[:, :1])
    idx_pad = jnp.zeros((mp_pad, K), jnp.int32).at[:mp].set(idx_dup)
    count_pad = jnp.zeros((mp_pad,), jnp.int32).at[:mp].set(count)

    nb = (count_pad + (CH - 1)) // CH
    nb_w = nb.reshape(NW, PTS)
    cs = jnp.cumsum(nb_w, axis=1)
    total = cs[:, -1]
    pos = jnp.arange(CAP, dtype=jnp.int32)
    widx = jnp.broadcast_to(jnp.arange(NW, dtype=jnp.int32)[:, None],
                            (NW, PTS))
    ind = jnp.zeros((NW, CAP + 1), jnp.int32).at[
        widx, jnp.minimum(cs, CAP)].add(1)
    pt = jnp.cumsum(ind[:, :CAP], axis=1)
    pt_c = jnp.minimum(pt, PTS - 1).astype(jnp.int32)
    cs_excl = cs - nb_w
    sspos = jnp.where(nb_w > 0, cs_excl, CAP)
    starts = jnp.zeros((NW, CAP + 1), jnp.int32).at[
        widx, jnp.minimum(sspos, CAP)].add(cs_excl)
    start = jax.lax.cummax(starts[:, :CAP], axis=1)
    sub = (pos[None, :] - start).astype(jnp.int32)
    valid = pos[None, :] < total[:, None]

    csrc = pt_c * K + sub * CH
    meta = jnp.where(valid, (csrc << 10) | pt_c, PTS)
    meta = jnp.concatenate(
        [meta, jnp.full((NW, CAPM - CAP), PTS, jnp.int32)], axis=1)
    meta_rep = jnp.broadcast_to(meta[:, :, None], (NW, CAPM, 8)).reshape(-1)
    glen = ((total + CPD * NBUF - 1) // (CPD * NBUF)).astype(jnp.int32)
    glen_rep = jnp.broadcast_to(glen[:, None], (NW, 128)).reshape(-1)

    grid_kernel = pl.kernel(
        _pool_body,
        out_type=jax.ShapeDtypeStruct((mp_pad * C,), jnp.float32),
        mesh=plsc.VectorSubcoreMesh(core_axis_name="c", subcore_axis_name="s"),
        scratch_types=[
            pltpu.VMEM((PTS * K + 8,), jnp.int32),
            pltpu.VMEM((CAPM * 8,), jnp.int32),
            pltpu.VMEM((128,), jnp.int32),
            pltpu.VMEM((NBUF * BSTR,), jnp.int32),
            pltpu.VMEM((NBUF, CPD * CH, C), jnp.float32),
            pltpu.VMEM(((PTS + 1) * C,), jnp.float32),
            pltpu.SemaphoreType.DMA((NBUF,)),
        ],
    )
    out = grid_kernel(inputs, idx_pad.reshape(-1), meta_rep, glen_rep)
    return out.reshape(mp_pad, C)[:mp]
